# Initial kernel scaffold; baseline (speedup 1.0000x reference)
import jax
import jax.numpy as jnp
from jax.experimental import pallas as pl


def kernel(q_len, k_len, bias):
    raise NotImplementedError("write your pallas kernel here")


# speedup vs baseline: 42.4201x; 42.4201x over previous
import jax
import jax.numpy as jnp
from jax import lax
from jax.experimental import pallas as pl
from jax.experimental.pallas import tpu as pltpu
from jax.experimental.pallas import tpu_sc as plsc

H = 16
Q = 2048
K = 2048
T = 257
D = (T - 1) // 2
TPAD = 264
GPAD = 4112
NSHIFT = 8
ROWS_PER_TILE = Q // 2
FLIGHT = 16

M = K - 1
BAND_LO = M - D + 1 - 15
FILL_HI = 1920
FILL_LO = 2176


def _bcast_lane(v, lane):
    idx = jnp.full((16, 1), lane, jnp.int32)
    dnums = lax.GatherDimensionNumbers(
        offset_dims=(), collapsed_slice_dims=(0,), start_index_map=(0,)
    )
    return lax.gather(v, idx, dnums, slice_sizes=(1,),
                      mode=lax.GatherScatterMode.PROMISE_IN_BOUNDS)


def _rpb_sc(bias_hbm, out_hbm, col_v, *gs_and_sem):
    gs = gs_and_sem[:NSHIFT]
    sem = gs_and_sem[NSHIFT]
    c = lax.axis_index("c")
    s = lax.axis_index("s")
    pltpu.sync_copy(bias_hbm.at[s], col_v)

    c_hi = _bcast_lane(col_v[pl.ds(248, 16)], 8)
    c_lo = _bcast_lane(col_v[pl.ds(0, 16)], 0)

    for a in range(NSHIFT):
        g_a = gs[a]

        def fill_hi(u, carry, g_a=g_a):
            g_a[pl.ds(u * 16, 16)] = c_hi
            return carry

        def fill_lo(u, carry, g_a=g_a):
            g_a[pl.ds(FILL_LO + u * 16, 16)] = c_lo
            return carry

        lax.fori_loop(0, FILL_HI // 16, fill_hi, 0)
        lax.fori_loop(0, (GPAD - FILL_LO) // 16, fill_lo, 0)

    g0 = gs[0]
    for w0 in range(FILL_HI, FILL_LO, 16):
        g0[pl.ds(w0, 16)] = lax.rev(col_v[pl.ds(2160 - w0, 16)], (0,))

    for a in range(1, NSHIFT):
        g_a = gs[a]
        for w0 in range(BAND_LO, FILL_LO, 16):
            g_a[pl.ds(w0, 16)] = g0[pl.ds(w0 + a, 16)]

    def blk(b, carry):
        r0 = c * ROWS_PER_TILE + b * FLIGHT
        descs = []
        for t in range(FLIGHT):
            i = r0 + t
            a = (M - t) % NSHIFT
            off = pl.multiple_of(M - i - a, NSHIFT)
            dst_off = pl.multiple_of((s * Q + i) * K, NSHIFT)
            descs.append(
                pltpu.async_copy(
                    gs[a].at[pl.ds(off, K)], out_hbm.at[pl.ds(dst_off, K)], sem
                )
            )
        for dcp in descs:
            dcp.wait()
        return carry

    lax.fori_loop(0, ROWS_PER_TILE // FLIGHT, blk, 0)


@jax.jit
def _launch(bias):
    bias_t = jnp.pad(bias.T, ((0, 0), (0, TPAD - T)))
    fn = pl.kernel(
        _rpb_sc,
        mesh=plsc.VectorSubcoreMesh(core_axis_name="c", subcore_axis_name="s"),
        out_type=jax.ShapeDtypeStruct((H * Q * K,), jnp.float32),
        scratch_types=[pltpu.VMEM((TPAD,), jnp.float32)]
        + [pltpu.VMEM((GPAD,), jnp.float32) for _ in range(NSHIFT)]
        + [pltpu.SemaphoreType.DMA],
    )
    return fn(bias_t).reshape(H, Q, K)


def kernel(q_len, k_len, bias):
    return _launch(bias)
